# Initial kernel scaffold; baseline (speedup 1.0000x reference)
import jax
import jax.numpy as jnp
from jax.experimental import pallas as pl


def kernel(tokens, W1, b1, W2, b2):
    raise NotImplementedError("write your pallas kernel here")


# speedup vs baseline: 4.9765x; 4.9765x over previous
import jax
import jax.numpy as jnp
from jax import lax
from jax.experimental import pallas as pl
from jax.experimental.pallas import tpu as pltpu
from jax.experimental.pallas import tpu_sc as plsc

DIM = 768
HID = DIM // 2
NTOK = 8192
NBATCH = 4
KEEP = 4096
NROWS = NBATCH * NTOK
OUT_ROWS = NBATCH * KEEP
ROW_TILE = 1024
NTILES = NROWS // ROW_TILE
ROWS_PER_BATCH = NTOK // ROW_TILE
TCHUNK = 1024

NC = 2
NSUB = 16
NWORK = NC * NSUB
OUT_PER_W = OUT_ROWS // NWORK
GCHUNK = 128
NGC = OUT_PER_W // GCHUNK


def _score_body(tok_ref, w1_ref, b1_ref, w2_ref, b2_ref, out_ref):
    h = jnp.dot(tok_ref[...], w1_ref[...], preferred_element_type=jnp.float32)
    h = h + b1_ref[...]
    h = 0.5 * h * (1.0 + lax.erf(h * jnp.float32(0.7071067811865476)))
    s = jnp.dot(h, w2_ref[...], preferred_element_type=jnp.float32)
    out_ref[...] = s + b2_ref[...]


def _scores_tc(tok2d, W1, b1, W2, b2, interpret=False):
    return pl.pallas_call(
        _score_body,
        grid=(NTILES,),
        in_specs=[
            pl.BlockSpec((ROW_TILE, DIM), lambda i: (i, 0)),
            pl.BlockSpec((DIM, HID), lambda i: (0, 0)),
            pl.BlockSpec((1, HID), lambda i: (0, 0)),
            pl.BlockSpec((HID, 1), lambda i: (0, 0)),
            pl.BlockSpec((1, 1), lambda i: (0, 0)),
        ],
        out_specs=pl.BlockSpec((ROW_TILE, 1), lambda i: (i, 0)),
        out_shape=jax.ShapeDtypeStruct((NROWS, 1), jnp.float32),
        interpret=interpret,
    )(tok2d, W1, b1.reshape(1, HID), W2, b2.reshape(1, 1))


def _select_body(s_ref, src_ref):
    ir = lax.broadcasted_iota(jnp.int32, (ROW_TILE, ROW_TILE), 0)
    ic = lax.broadcasted_iota(jnp.int32, (ROW_TILE, ROW_TILE), 1)
    upper = (ir <= ic).astype(jnp.float32)
    lr = lax.broadcasted_iota(jnp.int32, (ROWS_PER_BATCH, ROWS_PER_BATCH), 0)
    lc = lax.broadcasted_iota(jnp.int32, (ROWS_PER_BATCH, ROWS_PER_BATCH), 1)
    strict_lower = (lc < lr).astype(jnp.float32)

    def cumsum_rows(x):
        within = jnp.dot(x, upper, preferred_element_type=jnp.float32)
        row_off = jnp.sum(
            jnp.dot(strict_lower, x, preferred_element_type=jnp.float32),
            axis=1, keepdims=True)
        return within + row_off

    tcol = lax.broadcasted_iota(jnp.int32, (TCHUNK, 1, 1), 0)

    for b in range(NBATCH):
        s = s_ref[b * ROWS_PER_BATCH:(b + 1) * ROWS_PER_BATCH, :]
        u = lax.bitcast_convert_type(s, jnp.uint32)
        sgn = u >> jnp.uint32(31)
        key = u ^ (jnp.uint32(0x80000000) + sgn * jnp.uint32(0x7FFFFFFF))

        def bis(t, T):
            bit = jnp.uint32(31) - t.astype(jnp.uint32)
            cand = T | (jnp.uint32(1) << bit)
            cnt = jnp.sum((key >= cand).astype(jnp.int32))
            return jnp.where(cnt >= KEEP, cand, T)

        T = lax.fori_loop(0, 32, bis, jnp.uint32(0))

        gt = key > T
        eq = key == T
        need = KEEP - jnp.sum(gt.astype(jnp.int32))
        eq_cum = cumsum_rows(eq.astype(jnp.float32)).astype(jnp.int32)
        kept = gt | (eq & (eq_cum <= need))
        rank_incl = cumsum_rows(kept.astype(jnp.float32)).astype(jnp.int32)

        r3 = rank_incl[None, :, :]
        for tc in range(KEEP // TCHUNK):
            tval = tcol + tc * TCHUNK
            cnts = jnp.sum((r3 <= tval).astype(jnp.int32), axis=(1, 2),
                           keepdims=False)
            src_ref[b * KEEP + tc * TCHUNK:
                    b * KEEP + (tc + 1) * TCHUNK, :] = (
                cnts.reshape(TCHUNK, 1) + b * NTOK)


def _select_tc(scores, interpret=False):
    return pl.pallas_call(
        _select_body,
        out_shape=jax.ShapeDtypeStruct((OUT_ROWS, 1), jnp.int32),
        interpret=interpret,
    )(scores)


def _diag_body(s_ref, keep_ref):
    for b in range(NBATCH):
        s = s_ref[b * ROWS_PER_BATCH:(b + 1) * ROWS_PER_BATCH, :]
        u = lax.bitcast_convert_type(s, jnp.uint32)
        sgn = u >> jnp.uint32(31)
        key = u ^ (jnp.uint32(0x80000000) + sgn * jnp.uint32(0x7FFFFFFF))

        def bis(t, T):
            bit = jnp.uint32(31) - t.astype(jnp.uint32)
            cand = T | (jnp.uint32(1) << bit)
            cnt = jnp.sum((key >= cand).astype(jnp.int32))
            return jnp.where(cnt >= KEEP, cand, T)

        T = lax.fori_loop(0, 32, bis, jnp.uint32(0))
        gt = key > T
        ir = lax.broadcasted_iota(jnp.int32, (ROW_TILE, ROW_TILE), 0)
        ic = lax.broadcasted_iota(jnp.int32, (ROW_TILE, ROW_TILE), 1)
        upper = (ir <= ic).astype(jnp.float32)
        lr = lax.broadcasted_iota(jnp.int32, (ROWS_PER_BATCH, ROWS_PER_BATCH), 0)
        lc = lax.broadcasted_iota(jnp.int32, (ROWS_PER_BATCH, ROWS_PER_BATCH), 1)
        strict_lower = (lc < lr).astype(jnp.float32)
        x = gt.astype(jnp.float32)
        within = jnp.dot(x, upper, preferred_element_type=jnp.float32)
        row_off = jnp.sum(
            jnp.dot(strict_lower, x, preferred_element_type=jnp.float32),
            axis=1, keepdims=True)
        keep_ref[b * ROWS_PER_BATCH:(b + 1) * ROWS_PER_BATCH, :] = (
            within + row_off).astype(jnp.int32)


def _diag_tc(scores):
    return pl.pallas_call(
        _diag_body,
        out_shape=jax.ShapeDtypeStruct((NTILES, ROW_TILE), jnp.int32),
    )(scores)


def _sc_body(tok_hbm, sidx_hbm, out_hbm, idx_v, rows, sem):
    wid = lax.axis_index("s") * NC + lax.axis_index("c")
    base = wid * OUT_PER_W
    pltpu.sync_copy(sidx_hbm.at[pl.ds(base, OUT_PER_W)], idx_v)

    def gbody(j, _):
        pltpu.async_copy(
            tok_hbm.at[idx_v.at[pl.ds(GCHUNK * j, GCHUNK)]], rows, sem).wait()
        pltpu.sync_copy(rows, out_hbm.at[pl.ds(base + GCHUNK * j, GCHUNK)])
        return 0

    lax.fori_loop(0, NGC, gbody, 0)


def _sc_gather(tok2d, sidx_flat):
    mesh = plsc.VectorSubcoreMesh(core_axis_name="c", subcore_axis_name="s")
    f = pl.kernel(
        _sc_body,
        out_type=jax.ShapeDtypeStruct((OUT_ROWS, DIM), jnp.float32),
        mesh=mesh,
        scratch_types=[
            pltpu.VMEM((OUT_PER_W,), jnp.int32),
            pltpu.VMEM((GCHUNK, DIM), jnp.float32),
            pltpu.SemaphoreType.DMA,
        ],
    )
    return f(tok2d, sidx_flat)


def kernel(tokens, W1, b1, W2, b2):
    tok2d = tokens.reshape(NROWS, DIM)
    s = _scores_tc(tok2d, W1, b1, W2, b2)
    sidx = _select_tc(s.reshape(NTILES, ROW_TILE))
    out = _sc_gather(tok2d, sidx.reshape(OUT_ROWS))
    return out.reshape(NBATCH, KEEP, DIM)
